# Initial kernel scaffold; baseline (speedup 1.0000x reference)
import functools

import jax
import jax.numpy as jnp
from jax import lax
from jax.experimental import pallas as pl
from jax.experimental.pallas import tpu as pltpu
from jax.experimental.pallas import tpu_sc as plsc

VOCAB = 100000
D = 128
B = 16384
NNEG = 20

NC, NS, L = 2, 16, 16
NW = NC * NS
BW = B // NW
NB = 32
NCHUNK = BW // NB
NEG_ROWS = NB * NNEG
GCH = 64
NGN = NEG_ROWS // GCH
DK = D // L


def _tree_sum(vs):
    while len(vs) > 1:
        nxt = [vs[i] + vs[i + 1] for i in range(0, len(vs) - 1, 2)]
        if len(vs) % 2:
            nxt.append(vs[-1])
        vs = nxt
    return vs[0]


def _sc_scores(centers, contexts, neg_flat, in_w, out_w):
    mesh = plsc.VectorSubcoreMesh(core_axis_name="c", subcore_axis_name="s")

    @functools.partial(
        pl.kernel,
        out_type=(jax.ShapeDtypeStruct((B,), jnp.float32),
                  jax.ShapeDtypeStruct((B * NNEG,), jnp.float32)),
        mesh=mesh,
        scratch_types=[
            pltpu.VMEM((NB,), jnp.int32),
            pltpu.VMEM((NB,), jnp.int32),
            pltpu.VMEM((NGN, GCH), jnp.int32),
            pltpu.VMEM((NB, D), jnp.float32),
            pltpu.VMEM((NB, D), jnp.float32),
            pltpu.VMEM((NEG_ROWS, D), jnp.float32),
            pltpu.VMEM((NB,), jnp.float32),
            pltpu.VMEM((NEG_ROWS,), jnp.float32),
            pltpu.SemaphoreType.DMA,
        ],
    )
    def k(centers_hbm, contexts_hbm, negs_hbm, in_w_hbm, out_w_hbm,
          pos_hbm, nsc_hbm,
          cidx, xidx, nidx, crows, xrows, nrows, posbuf, negbuf, sem):
        wid = lax.axis_index("s") * NC + lax.axis_index("c")
        base0 = wid * BW
        lane0 = jnp.arange(L, dtype=jnp.int32) == 0

        @pl.loop(0, NCHUNK)
        def chunk(g):
            base = base0 + g * NB
            pltpu.sync_copy(centers_hbm.at[pl.ds(base, NB)], cidx)
            pltpu.sync_copy(contexts_hbm.at[pl.ds(base, NB)], xidx)
            for j in range(NGN):
                pltpu.sync_copy(
                    negs_hbm.at[pl.ds(base * NNEG + j * GCH, GCH)], nidx.at[j])
            descs = [pltpu.async_copy(in_w_hbm.at[cidx], crows, sem),
                     pltpu.async_copy(out_w_hbm.at[xidx], xrows, sem)]
            for j in range(NGN):
                descs.append(pltpu.async_copy(
                    out_w_hbm.at[nidx.at[j]],
                    nrows.at[pl.ds(j * GCH, GCH)], sem))
            for dsc in descs:
                dsc.wait()

            @pl.loop(0, NB)
            def belem(b):
                c = [crows[b, pl.ds(kk * L, L)] for kk in range(DK)]
                p = [c[kk] * xrows[b, pl.ds(kk * L, L)] for kk in range(DK)]
                s = jnp.sum(_tree_sum(p))
                plsc.store_scatter(
                    posbuf, [jnp.full((L,), b, jnp.int32)],
                    jnp.full((L,), s, jnp.float32), mask=lane0)
                for n in range(NNEG):
                    r = b * NNEG + n
                    q = [c[kk] * nrows[r, pl.ds(kk * L, L)]
                         for kk in range(DK)]
                    sn = jnp.sum(_tree_sum(q))
                    plsc.store_scatter(
                        negbuf, [jnp.full((L,), r, jnp.int32)],
                        jnp.full((L,), sn, jnp.float32), mask=lane0)

            pltpu.sync_copy(posbuf, pos_hbm.at[pl.ds(base, NB)])
            pltpu.sync_copy(negbuf, nsc_hbm.at[pl.ds(base * NNEG, NEG_ROWS)])

    return k(centers, contexts, neg_flat, in_w, out_w)


def _loss_body(p_ref, n_ref, o_ref):
    p = p_ref[...]
    n = n_ref[...]
    lp = jnp.minimum(p, 0.0) - jnp.log1p(jnp.exp(-jnp.abs(p)))
    ln = jnp.minimum(-n, 0.0) - jnp.log1p(jnp.exp(-jnp.abs(n)))
    o_ref[0, 0] = -(jnp.sum(lp) + jnp.sum(ln)) / B


def _loss(pos, nsc):
    out = pl.pallas_call(
        _loss_body,
        out_shape=jax.ShapeDtypeStruct((1, 1), jnp.float32),
        out_specs=pl.BlockSpec(memory_space=pltpu.SMEM),
    )(pos.reshape(B // 128, 128), nsc.reshape(B * NNEG // 128, 128))
    return out[0, 0]


def kernel(centers, contexts, negatives, in_embed_w, out_embed_w):
    centers = centers.astype(jnp.int32)
    contexts = contexts.astype(jnp.int32)
    neg_flat = negatives.astype(jnp.int32).reshape(B * NNEG)
    pos, nsc = _sc_scores(centers, contexts, neg_flat,
                          in_embed_w, out_embed_w)
    return _loss(pos, nsc)


# speedup vs baseline: 5.6803x; 5.6803x over previous
import functools

import jax
import jax.numpy as jnp
from jax import lax
from jax.experimental import pallas as pl
from jax.experimental.pallas import tpu as pltpu
from jax.experimental.pallas import tpu_sc as plsc

VOCAB = 100000
D = 128
B = 16384
NNEG = 20
NSLOT = NNEG + 1

NC, NS, L = 2, 16, 16
NW = NC * NS
BW = B // NW
NB = 32
NCHUNK = BW // NB
NEG_ROWS = NB * NNEG
GCH = 64
NGN = NEG_ROWS // GCH
DK = D // L


def _tree_sum(vs):
    while len(vs) > 1:
        nxt = [vs[i] + vs[i + 1] for i in range(0, len(vs) - 1, 2)]
        if len(vs) % 2:
            nxt.append(vs[-1])
        vs = nxt
    return vs[0]


def _lane_sum(v, perms):
    for p in perms:
        v = v + v.at[p].get(mode="promise_in_bounds")
    return v


def _sc_scores(centers, contexts, neg_flat, in_w, out_w):
    mesh = plsc.VectorSubcoreMesh(core_axis_name="c", subcore_axis_name="s")

    @functools.partial(
        pl.kernel,
        out_type=jax.ShapeDtypeStruct((B // NB, NSLOT * NB), jnp.float32),
        mesh=mesh,
        scratch_types=[
            pltpu.VMEM((NB,), jnp.int32),
            pltpu.VMEM((NB,), jnp.int32),
            pltpu.VMEM((NGN, GCH), jnp.int32),
            pltpu.VMEM((NB, D), jnp.float32),
            pltpu.VMEM((NB, D), jnp.float32),
            pltpu.VMEM((NEG_ROWS, D), jnp.float32),
            pltpu.VMEM((NSLOT * NB,), jnp.float32),
            pltpu.SemaphoreType.DMA,
        ],
    )
    def k(centers_hbm, contexts_hbm, negs_hbm, in_w_hbm, out_w_hbm,
          sc_hbm, cidx, xidx, nidx, crows, xrows, nrows, sbuf, sem):
        wid = lax.axis_index("s") * NC + lax.axis_index("c")
        base0 = wid * BW
        lanes = jnp.arange(L, dtype=jnp.int32)
        perms = [lanes ^ kk for kk in (8, 4, 2, 1)]
        zero = jnp.zeros((L,), jnp.float32)

        @pl.loop(0, NCHUNK)
        def chunk(g):
            base = base0 + g * NB
            pltpu.sync_copy(centers_hbm.at[pl.ds(base, NB)], cidx)
            pltpu.sync_copy(contexts_hbm.at[pl.ds(base, NB)], xidx)
            for j in range(NGN):
                pltpu.sync_copy(
                    negs_hbm.at[pl.ds(base * NNEG + j * GCH, GCH)], nidx.at[j])
            descs = [pltpu.async_copy(in_w_hbm.at[cidx], crows, sem),
                     pltpu.async_copy(out_w_hbm.at[xidx], xrows, sem)]
            for j in range(NGN):
                descs.append(pltpu.async_copy(
                    out_w_hbm.at[nidx.at[j]],
                    nrows.at[pl.ds(j * GCH, GCH)], sem))
            for dsc in descs:
                dsc.wait()

            @pl.loop(0, NB, init_carry=(zero,) * NSLOT)
            def belem(b, accs):
                lane_m = lanes == (b % L)
                c = [crows[b, pl.ds(kk * L, L)] for kk in range(DK)]
                p = [c[kk] * xrows[b, pl.ds(kk * L, L)] for kk in range(DK)]
                s = _lane_sum(_tree_sum(p), perms)
                new = [jnp.where(lane_m, s, accs[0])]
                for n in range(NNEG):
                    r = b * NNEG + n
                    q = [c[kk] * nrows[r, pl.ds(kk * L, L)]
                         for kk in range(DK)]
                    sn = _lane_sum(_tree_sum(q), perms)
                    new.append(jnp.where(lane_m, sn, accs[n + 1]))

                @pl.when(b % L == L - 1)
                def _flush():
                    off = (b // L) * L
                    for n in range(NSLOT):
                        sbuf[pl.ds(n * NB + off, L)] = new[n]

                return tuple(new)

            pltpu.sync_copy(sbuf, sc_hbm.at[wid * NCHUNK + g])

    return k(centers, contexts, neg_flat, in_w, out_w)


def _loss_body(s_ref, o_ref):
    s = s_ref[...]
    col = lax.broadcasted_iota(jnp.int32, s.shape, 1)
    x = jnp.where(col < NB, s, -s)
    l = jnp.minimum(x, 0.0) - jnp.log1p(jnp.exp(-jnp.abs(x)))
    o_ref[0, 0] = -jnp.sum(l) / B


def _loss(scores):
    out = pl.pallas_call(
        _loss_body,
        out_shape=jax.ShapeDtypeStruct((1, 1), jnp.float32),
        out_specs=pl.BlockSpec(memory_space=pltpu.SMEM),
    )(scores)
    return out[0, 0]


def kernel(centers, contexts, negatives, in_embed_w, out_embed_w):
    centers = centers.astype(jnp.int32)
    contexts = contexts.astype(jnp.int32)
    neg_flat = negatives.astype(jnp.int32).reshape(B * NNEG)
    scores = _sc_scores(centers, contexts, neg_flat,
                        in_embed_w, out_embed_w)
    return _loss(scores)


# speedup vs baseline: 7.4666x; 1.3145x over previous
import functools

import jax
import jax.numpy as jnp
from jax import lax
from jax.experimental import pallas as pl
from jax.experimental.pallas import tpu as pltpu
from jax.experimental.pallas import tpu_sc as plsc

VOCAB = 100000
D = 128
B = 16384
NNEG = 20
NSLOT = NNEG + 1

NC, NS, L = 2, 16, 16
NW = NC * NS
BW = B // NW
NB = 16
NCHUNK = BW // NB
NEG_ROWS = NB * NNEG
GCH = 64
NGN = NEG_ROWS // GCH
DK = D // L


def _tree_sum(vs):
    while len(vs) > 1:
        nxt = [vs[i] + vs[i + 1] for i in range(0, len(vs) - 1, 2)]
        if len(vs) % 2:
            nxt.append(vs[-1])
        vs = nxt
    return vs[0]


def _lane_sum(v, perms):
    for p in perms:
        v = v + v.at[p].get(mode="promise_in_bounds")
    return v


def _sc_scores(centers, contexts, neg_flat, in_w, out_w):
    mesh = plsc.VectorSubcoreMesh(core_axis_name="c", subcore_axis_name="s")

    slot_scratch = [
        pltpu.VMEM((NB,), jnp.int32),
        pltpu.VMEM((NB,), jnp.int32),
        pltpu.VMEM((NGN, GCH), jnp.int32),
        pltpu.VMEM((NB, D), jnp.float32),
        pltpu.VMEM((NB, D), jnp.float32),
        pltpu.VMEM((NEG_ROWS, D), jnp.float32),
        pltpu.VMEM((NSLOT * NB,), jnp.float32),
        pltpu.SemaphoreType.DMA,
    ]

    @functools.partial(
        pl.kernel,
        out_type=jax.ShapeDtypeStruct((B // NB, NSLOT * NB), jnp.float32),
        mesh=mesh,
        scratch_types=slot_scratch + slot_scratch,
    )
    def k(centers_hbm, contexts_hbm, negs_hbm, in_w_hbm, out_w_hbm,
          sc_hbm, *scratch):
        slots = (scratch[:8], scratch[8:])
        wid = lax.axis_index("s") * NC + lax.axis_index("c")
        base0 = wid * BW
        lanes = jnp.arange(L, dtype=jnp.int32)
        perms = [lanes ^ kk for kk in (8, 4, 2, 1)]
        zero = jnp.zeros((L,), jnp.float32)

        def issue(g, slot):
            cidx, xidx, nidx, crows, xrows, nrows, _, sem = slot
            base = base0 + g * NB
            pltpu.sync_copy(centers_hbm.at[pl.ds(base, NB)], cidx)
            pltpu.sync_copy(contexts_hbm.at[pl.ds(base, NB)], xidx)
            for j in range(NGN):
                pltpu.sync_copy(
                    negs_hbm.at[pl.ds(base * NNEG + j * GCH, GCH)], nidx.at[j])
            pltpu.async_copy(in_w_hbm.at[cidx], crows, sem)
            pltpu.async_copy(out_w_hbm.at[xidx], xrows, sem)
            for j in range(NGN):
                pltpu.async_copy(out_w_hbm.at[nidx.at[j]],
                                 nrows.at[pl.ds(j * GCH, GCH)], sem)

        def drain(slot):
            cidx, xidx, nidx, crows, xrows, nrows, _, sem = slot
            pltpu.make_async_copy(in_w_hbm.at[cidx], crows, sem).wait()
            pltpu.make_async_copy(out_w_hbm.at[xidx], xrows, sem).wait()
            for j in range(NGN):
                pltpu.make_async_copy(out_w_hbm.at[nidx.at[j]],
                                      nrows.at[pl.ds(j * GCH, GCH)],
                                      sem).wait()

        def compute(g, slot):
            _, _, _, crows, xrows, nrows, sbuf, _ = slot

            @pl.loop(0, NB, init_carry=(zero,) * NSLOT)
            def belem(b, accs):
                lane_m = lanes == b
                c = [crows[b, pl.ds(kk * L, L)] for kk in range(DK)]
                p = [c[kk] * xrows[b, pl.ds(kk * L, L)] for kk in range(DK)]
                s = _lane_sum(_tree_sum(p), perms)
                new = [jnp.where(lane_m, s, accs[0])]
                for n in range(NNEG):
                    r = b * NNEG + n
                    q = [c[kk] * nrows[r, pl.ds(kk * L, L)]
                         for kk in range(DK)]
                    sn = _lane_sum(_tree_sum(q), perms)
                    new.append(jnp.where(lane_m, sn, accs[n + 1]))
                return tuple(new)

            for n in range(NSLOT):
                sbuf[pl.ds(n * NB, NB)] = belem[n]
            pltpu.sync_copy(sbuf, sc_hbm.at[wid * NCHUNK + g])

        issue(0, slots[0])

        @pl.loop(0, NCHUNK, step=2)
        def pair(g):
            issue(g + 1, slots[1])
            drain(slots[0])
            compute(g, slots[0])

            @pl.when(g + 2 < NCHUNK)
            def _():
                issue(g + 2, slots[0])

            drain(slots[1])
            compute(g + 1, slots[1])

    return k(centers, contexts, neg_flat, in_w, out_w)


def _loss_body(s_ref, o_ref):
    s = s_ref[...]
    col = lax.broadcasted_iota(jnp.int32, s.shape, 1)
    x = jnp.where(col < NB, s, -s)
    l = jnp.minimum(x, 0.0) - jnp.log1p(jnp.exp(-jnp.abs(x)))
    o_ref[0, 0] = -jnp.sum(l) / B


def _loss(scores):
    out = pl.pallas_call(
        _loss_body,
        out_shape=jax.ShapeDtypeStruct((1, 1), jnp.float32),
        out_specs=pl.BlockSpec(memory_space=pltpu.SMEM),
    )(scores)
    return out[0, 0]


def kernel(centers, contexts, negatives, in_embed_w, out_embed_w):
    centers = centers.astype(jnp.int32)
    contexts = contexts.astype(jnp.int32)
    neg_flat = negatives.astype(jnp.int32).reshape(B * NNEG)
    scores = _sc_scores(centers, contexts, neg_flat,
                        in_embed_w, out_embed_w)
    return _loss(scores)


# speedup vs baseline: 12.8693x; 1.7236x over previous
import functools

import jax
import jax.numpy as jnp
from jax import lax
from jax.experimental import pallas as pl
from jax.experimental.pallas import tpu as pltpu
from jax.experimental.pallas import tpu_sc as plsc

VOCAB = 100000
D = 128
B = 16384
NNEG = 20
NSLOT = NNEG + 1

NC, NS, L = 2, 16, 16
NW = NC * NS
BW = B // NW
NB = 16
NCHUNK = BW // NB
NEG_ROWS = NB * NNEG
GCH = 64
NGN = NEG_ROWS // GCH
DK = D // L


def _tree_sum(vs):
    while len(vs) > 1:
        nxt = [vs[i] + vs[i + 1] for i in range(0, len(vs) - 1, 2)]
        if len(vs) % 2:
            nxt.append(vs[-1])
        vs = nxt
    return vs[0]


def _lane_sum(v, perms):
    for p in perms:
        v = v + v.at[p].get(mode="promise_in_bounds")
    return v


def _sc_scores(centers, contexts, neg_flat, in_w, out_w):
    mesh = plsc.VectorSubcoreMesh(core_axis_name="c", subcore_axis_name="s")

    slot_scratch = [
        pltpu.VMEM((NB, D), jnp.float32),
        pltpu.VMEM((NB, D), jnp.float32),
        pltpu.VMEM((NEG_ROWS, D), jnp.float32),
        pltpu.VMEM((NSLOT * NB,), jnp.float32),
        pltpu.SemaphoreType.DMA,
        pltpu.SemaphoreType.DMA,
    ]

    @functools.partial(
        pl.kernel,
        out_type=jax.ShapeDtypeStruct((B // NB, NSLOT * NB), jnp.float32),
        mesh=mesh,
        scratch_types=[
            pltpu.VMEM((BW,), jnp.int32),
            pltpu.VMEM((BW,), jnp.int32),
            pltpu.VMEM((BW * NNEG,), jnp.int32),
            pltpu.SemaphoreType.DMA,
        ] + slot_scratch + slot_scratch,
    )
    def k(centers_hbm, contexts_hbm, negs_hbm, in_w_hbm, out_w_hbm,
          sc_hbm, cidx, xidx, nidx, isem, *scratch):
        slots = (scratch[:6], scratch[6:])
        wid = lax.axis_index("s") * NC + lax.axis_index("c")
        base0 = wid * BW
        lanes = jnp.arange(L, dtype=jnp.int32)
        perms = [lanes ^ kk for kk in (8, 4, 2, 1)]
        zero = jnp.zeros((L,), jnp.float32)

        stage = [
            pltpu.async_copy(centers_hbm.at[pl.ds(base0, BW)], cidx, isem),
            pltpu.async_copy(contexts_hbm.at[pl.ds(base0, BW)], xidx, isem),
            pltpu.async_copy(negs_hbm.at[pl.ds(base0 * NNEG, BW * NNEG)],
                             nidx, isem),
        ]
        for dsc in stage:
            dsc.wait()

        def issue(g, slot):
            crows, xrows, nrows, _, sem, _ = slot
            pltpu.async_copy(in_w_hbm.at[cidx.at[pl.ds(g * NB, NB)]],
                             crows, sem)
            pltpu.async_copy(out_w_hbm.at[xidx.at[pl.ds(g * NB, NB)]],
                             xrows, sem)
            for j in range(NGN):
                pltpu.async_copy(
                    out_w_hbm.at[nidx.at[pl.ds(g * NEG_ROWS + j * GCH, GCH)]],
                    nrows.at[pl.ds(j * GCH, GCH)], sem)

        def drain(g, slot):
            crows, xrows, nrows, _, sem, _ = slot
            pltpu.make_async_copy(in_w_hbm.at[cidx.at[pl.ds(g * NB, NB)]],
                                  crows, sem).wait()
            pltpu.make_async_copy(out_w_hbm.at[xidx.at[pl.ds(g * NB, NB)]],
                                  xrows, sem).wait()
            for j in range(NGN):
                pltpu.make_async_copy(
                    out_w_hbm.at[nidx.at[pl.ds(g * NEG_ROWS + j * GCH, GCH)]],
                    nrows.at[pl.ds(j * GCH, GCH)], sem).wait()

        def compute(g, slot):
            crows, xrows, nrows, sbuf, _, ssem = slot
            row = sc_hbm.at[wid * NCHUNK + g]

            @pl.loop(0, NB, init_carry=(zero,) * NSLOT)
            def accs(b, accs):
                lane_m = lanes == b
                c = [crows[b, pl.ds(kk * L, L)] for kk in range(DK)]
                p = [c[kk] * xrows[b, pl.ds(kk * L, L)] for kk in range(DK)]
                s = _lane_sum(_tree_sum(p), perms)
                new = [jnp.where(lane_m, s, accs[0])]
                for n in range(NNEG):
                    r = b * NNEG + n
                    q = [c[kk] * nrows[r, pl.ds(kk * L, L)]
                         for kk in range(DK)]
                    sn = _lane_sum(_tree_sum(q), perms)
                    new.append(jnp.where(lane_m, sn, accs[n + 1]))
                return tuple(new)

            @pl.when(g >= 2)
            def _():
                pltpu.make_async_copy(sbuf, row, ssem).wait()

            for n in range(NSLOT):
                sbuf[pl.ds(n * NB, NB)] = accs[n]
            pltpu.async_copy(sbuf, row, ssem)

        issue(0, slots[0])

        @pl.loop(0, NCHUNK, step=2)
        def pair(g):
            issue(g + 1, slots[1])
            drain(g, slots[0])
            compute(g, slots[0])

            @pl.when(g + 2 < NCHUNK)
            def _():
                issue(g + 2, slots[0])

            drain(g + 1, slots[1])
            compute(g + 1, slots[1])

        for g, slot in ((NCHUNK - 2, slots[0]), (NCHUNK - 1, slots[1])):
            pltpu.make_async_copy(slot[3], sc_hbm.at[wid * NCHUNK + g],
                                  slot[5]).wait()

    return k(centers, contexts, neg_flat, in_w, out_w)


def _loss_body(s_ref, o_ref):
    s = s_ref[...]
    col = lax.broadcasted_iota(jnp.int32, s.shape, 1)
    x = jnp.where(col < NB, s, -s)
    l = jnp.minimum(x, 0.0) - jnp.log1p(jnp.exp(-jnp.abs(x)))
    o_ref[0, 0] = -jnp.sum(l) / B


def _loss(scores):
    out = pl.pallas_call(
        _loss_body,
        out_shape=jax.ShapeDtypeStruct((1, 1), jnp.float32),
        out_specs=pl.BlockSpec(memory_space=pltpu.SMEM),
    )(scores)
    return out[0, 0]


def kernel(centers, contexts, negatives, in_embed_w, out_embed_w):
    centers = centers.astype(jnp.int32)
    contexts = contexts.astype(jnp.int32)
    neg_flat = negatives.astype(jnp.int32).reshape(B * NNEG)
    scores = _sc_scores(centers, contexts, neg_flat,
                        in_embed_w, out_embed_w)
    return _loss(scores)
